# Initial kernel scaffold; baseline (speedup 1.0000x reference)
import jax
import jax.numpy as jnp
from jax.experimental import pallas as pl


def kernel(x, space, memory, index):
    raise NotImplementedError("write your pallas kernel here")


# speedup vs baseline: 10.0141x; 10.0141x over previous
import functools

import jax
import jax.numpy as jnp
from jax import lax
from jax.experimental import pallas as pl
from jax.experimental.pallas import tpu as pltpu
from jax.experimental.pallas import tpu_sc as plsc

L = 65536
F = 512
NC = 2
NS = 16
NW = NC * NS
LANES = 16
CHUNK = L // NW
STEPS = CHUNK // LANES

_mesh = plsc.VectorSubcoreMesh(core_axis_name="c", subcore_axis_name="s")


@functools.partial(
    pl.kernel,
    out_type=(
        jax.ShapeDtypeStruct((NW, F), jnp.float32),
        jax.ShapeDtypeStruct((NW, LANES), jnp.float32),
        jax.ShapeDtypeStruct((NW, LANES), jnp.int32),
    ),
    mesh=_mesh,
    scratch_types=(
        pltpu.VMEM((CHUNK,), jnp.float32),
        pltpu.VMEM((LANES,), jnp.float32),
        pltpu.VMEM((LANES,), jnp.int32),
        pltpu.VMEM((LANES, F), jnp.float32),
        pltpu.SemaphoreType.DMA,
    ),
)
def _sc_argmax_rows(idx_hbm, mem_hbm, row_out, vmax_out, imax_out,
                    chunk_v, stage_f, stage_i, rows_v, sem):
    c = lax.axis_index("c")
    s = lax.axis_index("s")
    w = s * NC + c
    base = pl.multiple_of(w * CHUNK, CHUNK)
    pltpu.sync_copy(idx_hbm.at[pl.ds(base, CHUNK)], chunk_v)

    lane = lax.iota(jnp.int32, LANES)
    neg = jnp.full((LANES,), -jnp.inf, jnp.float32)

    def body(t, carry):
        cur_v, cur_i = carry
        off = pl.multiple_of(t * LANES, LANES)
        v = chunk_v[pl.ds(off, LANES)]
        pos = base + t * LANES + lane
        v = jnp.where(pos == 0, neg, v)
        upd = v >= cur_v
        return jnp.where(upd, v, cur_v), jnp.where(upd, pos, cur_i)

    cur_v, cur_i = lax.fori_loop(
        0, STEPS, body, (neg, jnp.full((LANES,), -1, jnp.int32)))

    wv = cur_v[0]
    wi = cur_i[0]
    for l in range(1, LANES):
        vl = cur_v[l]
        il = cur_i[l]
        upd = (vl > wv) | ((vl == wv) & (il > wi))
        wv = jnp.where(upd, vl, wv)
        wi = jnp.where(upd, il, wi)

    stage_i[...] = jnp.full((LANES,), wi, jnp.int32)
    pltpu.async_copy(mem_hbm.at[stage_i], rows_v, sem).wait()
    pltpu.sync_copy(rows_v.at[0], row_out.at[w])
    stage_f[...] = jnp.full((LANES,), wv, jnp.float32)
    pltpu.sync_copy(stage_f, vmax_out.at[w])
    pltpu.sync_copy(stage_i, imax_out.at[w])


def _tc_finalize_body(x_ref, rows_ref, vmax_ref, imax_ref,
                      out_row_ref, out_pri_ref):
    xv = x_ref[...]
    mx = jnp.max(xv)
    lse = jnp.log(jnp.sum(jnp.exp(xv - mx))) + mx
    pri = -jnp.sum(xv * (xv - lse))

    vmax = vmax_ref[...]
    imax = imax_ref[...]
    gv = jnp.max(vmax)
    gi = jnp.max(jnp.where(vmax == gv, imax, -1))
    hit = (vmax_ref[:, 0:1] == gv) & (imax_ref[:, 0:1] == gi)
    rows = rows_ref[...]
    rowbest = jnp.sum(jnp.where(hit, rows, 0.0), axis=0, keepdims=True)
    use_x = jnp.full((1, F), pri >= gv)
    out_row_ref[...] = jnp.where(use_x, xv, rowbest)
    out_pri_ref[...] = jnp.full((1, 1), pri, jnp.float32)


_tc_finalize = pl.pallas_call(
    _tc_finalize_body,
    out_shape=(
        jax.ShapeDtypeStruct((1, F), jnp.float32),
        jax.ShapeDtypeStruct((1, 1), jnp.float32),
    ),
)


def kernel(x, space, memory, index):
    idx_flat = index.reshape(L)
    mem2d = memory.reshape(L, F)
    rows, vmaxs, imaxs = _sc_argmax_rows(idx_flat, mem2d)
    return _tc_finalize(x, rows, vmaxs, imaxs)


# speedup vs baseline: 10.7401x; 1.0725x over previous
import functools

import jax
import jax.numpy as jnp
from jax import lax
from jax.experimental import pallas as pl
from jax.experimental.pallas import tpu as pltpu
from jax.experimental.pallas import tpu_sc as plsc

L = 65536
F = 512
NC = 2
NS = 16
NW = NC * NS
LANES = 16
CHUNK = L // NW
UNROLL = 8
SPAN = LANES * UNROLL
OUTER = CHUNK // SPAN

_mesh = plsc.VectorSubcoreMesh(core_axis_name="c", subcore_axis_name="s")


@functools.partial(
    pl.kernel,
    out_type=(
        jax.ShapeDtypeStruct((NW, F), jnp.float32),
        jax.ShapeDtypeStruct((NW, LANES), jnp.float32),
        jax.ShapeDtypeStruct((NW, LANES), jnp.int32),
    ),
    mesh=_mesh,
    scratch_types=(
        pltpu.VMEM((CHUNK,), jnp.float32),
        pltpu.VMEM((LANES,), jnp.float32),
        pltpu.VMEM((LANES,), jnp.int32),
        pltpu.VMEM((1, F), jnp.float32),
        pltpu.SemaphoreType.DMA,
    ),
)
def _sc_argmax_rows(idx_hbm, mem_hbm, row_out, vmax_out, imax_out,
                    chunk_v, stage_f, stage_i, row1_v, sem):
    c = lax.axis_index("c")
    s = lax.axis_index("s")
    w = s * NC + c
    base = pl.multiple_of(w * CHUNK, CHUNK)
    pltpu.sync_copy(idx_hbm.at[pl.ds(base, CHUNK)], chunk_v)

    lane = lax.iota(jnp.int32, LANES)
    neg = jnp.full((LANES,), -jnp.inf, jnp.float32)

    @pl.when(w == 0)
    def _():
        v0 = chunk_v[pl.ds(0, LANES)]
        chunk_v[pl.ds(0, LANES)] = jnp.where(lane == 0, neg, v0)

    lane_js = [jnp.full((LANES,), j * LANES, jnp.int32) + lane for j in range(UNROLL)]
    init_v = tuple(neg for _ in range(UNROLL))
    init_i = tuple(jnp.full((LANES,), -1, jnp.int32) for _ in range(UNROLL))

    def body(t, carry):
        vs = list(carry[:UNROLL])
        ps = list(carry[UNROLL:])
        tb = base + t * SPAN
        for j in range(UNROLL):
            off = pl.multiple_of(t * SPAN + j * LANES, LANES)
            v = chunk_v[pl.ds(off, LANES)]
            pos = tb + lane_js[j]
            upd = v >= vs[j]
            vs[j] = jnp.where(upd, v, vs[j])
            ps[j] = jnp.where(upd, pos, ps[j])
        return tuple(vs) + tuple(ps)

    acc = lax.fori_loop(0, OUTER, body, init_v + init_i)
    cur_v = acc[0]
    cur_i = acc[UNROLL]
    for j in range(1, UNROLL):
        vj = acc[j]
        ij = acc[UNROLL + j]
        upd = (vj > cur_v) | ((vj == cur_v) & (ij > cur_i))
        cur_v = jnp.where(upd, vj, cur_v)
        cur_i = jnp.where(upd, ij, cur_i)

    wv = cur_v[0]
    wi = cur_i[0]
    for l in range(1, LANES):
        vl = cur_v[l]
        il = cur_i[l]
        upd = (vl > wv) | ((vl == wv) & (il > wi))
        wv = jnp.where(upd, vl, wv)
        wi = jnp.where(upd, il, wi)

    stage_i[...] = jnp.full((LANES,), wi, jnp.int32)
    pltpu.async_copy(mem_hbm.at[stage_i.at[pl.ds(0, 1)]], row1_v, sem).wait()
    pltpu.sync_copy(row1_v.at[0], row_out.at[w])
    stage_f[...] = jnp.full((LANES,), wv, jnp.float32)
    pltpu.sync_copy(stage_f, vmax_out.at[w])
    pltpu.sync_copy(stage_i, imax_out.at[w])


def _tc_finalize_body(x_ref, rows_ref, vmax_ref, imax_ref, out_row_ref, out_pri_ref):
    xv = x_ref[...]
    mx = jnp.max(xv)
    lse = jnp.log(jnp.sum(jnp.exp(xv - mx))) + mx
    pri = -jnp.sum(xv * (xv - lse))

    vmax = vmax_ref[...]
    imax = imax_ref[...]
    gv = jnp.max(vmax)
    gi = jnp.max(jnp.where(vmax == gv, imax, -1))
    hit = (vmax[:, 0:1] == gv) & (imax[:, 0:1] == gi)
    rows = rows_ref[...]
    rowbest = jnp.sum(jnp.where(hit, rows, 0.0), axis=0, keepdims=True)
    use_x = jnp.full((1, F), pri >= gv)
    out_row_ref[...] = jnp.where(use_x, xv, rowbest)
    out_pri_ref[...] = jnp.full((1, 1), pri, jnp.float32)


_tc_finalize = pl.pallas_call(
    _tc_finalize_body,
    out_shape=(
        jax.ShapeDtypeStruct((1, F), jnp.float32),
        jax.ShapeDtypeStruct((1, 1), jnp.float32),
    ),
)


def kernel(x, space, memory, index):
    idx_flat = index.reshape(L)
    mem2d = memory.reshape(L, F)
    rows, vmaxs, imaxs = _sc_argmax_rows(idx_flat, mem2d)
    return _tc_finalize(x, rows, vmaxs, imaxs)


# speedup vs baseline: 10.7667x; 1.0025x over previous
import functools

import jax
import jax.numpy as jnp
from jax import lax
from jax.experimental import pallas as pl
from jax.experimental.pallas import tpu as pltpu
from jax.experimental.pallas import tpu_sc as plsc

L = 65536
F = 512
NC = 2
NS = 16
NW = NC * NS
LANES = 16
CHUNK = L // NW
UNROLL = 8
SPAN = LANES * UNROLL
OUTER = CHUNK // SPAN

_mesh = plsc.VectorSubcoreMesh(core_axis_name="c", subcore_axis_name="s")


@functools.partial(
    pl.kernel,
    out_type=(
        jax.ShapeDtypeStruct((NW, F), jnp.float32),
        jax.ShapeDtypeStruct((NW, LANES), jnp.float32),
    ),
    mesh=_mesh,
    scratch_types=(
        pltpu.VMEM((CHUNK,), jnp.float32),
        pltpu.VMEM((LANES,), jnp.float32),
        pltpu.VMEM((LANES,), jnp.int32),
        pltpu.VMEM((1, F), jnp.float32),
        pltpu.SemaphoreType.DMA,
    ),
)
def _sc_argmax_rows(idx_hbm, mem_hbm, row_out, vmax_out,
                    chunk_v, stage_f, stage_i, row1_v, sem):
    c = lax.axis_index("c")
    s = lax.axis_index("s")
    w = s * NC + c
    base = pl.multiple_of(w * CHUNK, CHUNK)
    pltpu.sync_copy(idx_hbm.at[pl.ds(base, CHUNK)], chunk_v)

    lane = lax.iota(jnp.int32, LANES)
    neg = jnp.full((LANES,), -jnp.inf, jnp.float32)

    @pl.when(w == 0)
    def _():
        v0 = chunk_v[pl.ds(0, LANES)]
        chunk_v[pl.ds(0, LANES)] = jnp.where(lane == 0, neg, v0)

    lane_js = [jnp.full((LANES,), j * LANES, jnp.int32) + lane for j in range(UNROLL)]
    init_v = tuple(neg for _ in range(UNROLL))
    init_i = tuple(jnp.full((LANES,), -1, jnp.int32) for _ in range(UNROLL))

    def body(t, carry):
        vs = list(carry[:UNROLL])
        ps = list(carry[UNROLL:])
        tb = base + t * SPAN
        for j in range(UNROLL):
            off = pl.multiple_of(t * SPAN + j * LANES, LANES)
            v = chunk_v[pl.ds(off, LANES)]
            pos = tb + lane_js[j]
            upd = v >= vs[j]
            vs[j] = jnp.where(upd, v, vs[j])
            ps[j] = jnp.where(upd, pos, ps[j])
        return tuple(vs) + tuple(ps)

    acc = lax.fori_loop(0, OUTER, body, init_v + init_i)
    cur_v = acc[0]
    cur_i = acc[UNROLL]
    for j in range(1, UNROLL):
        vj = acc[j]
        ij = acc[UNROLL + j]
        upd = (vj > cur_v) | ((vj == cur_v) & (ij > cur_i))
        cur_v = jnp.where(upd, vj, cur_v)
        cur_i = jnp.where(upd, ij, cur_i)

    wv = cur_v[0]
    wi = cur_i[0]
    for l in range(1, LANES):
        vl = cur_v[l]
        il = cur_i[l]
        upd = (vl > wv) | ((vl == wv) & (il > wi))
        wv = jnp.where(upd, vl, wv)
        wi = jnp.where(upd, il, wi)

    stage_i[...] = jnp.full((LANES,), wi, jnp.int32)
    gather = pltpu.async_copy(mem_hbm.at[stage_i.at[pl.ds(0, 1)]], row1_v, sem)
    stage_f[...] = jnp.full((LANES,), wv, jnp.float32)
    pltpu.sync_copy(stage_f, vmax_out.at[w])
    gather.wait()
    pltpu.sync_copy(row1_v.at[0], row_out.at[w])


def _tc_priority_body(x_ref, out_pri_ref):
    xv = x_ref[...]
    mx = jnp.max(xv)
    lse = jnp.log(jnp.sum(jnp.exp(xv - mx))) + mx
    pri = -jnp.sum(xv * (xv - lse))
    out_pri_ref[...] = jnp.full((1, 1), pri, jnp.float32)


_tc_priority = pl.pallas_call(
    _tc_priority_body,
    out_shape=jax.ShapeDtypeStruct((1, 1), jnp.float32),
)


def _tc_select_body(x_ref, pri_ref, rows_ref, vmax_ref, out_row_ref):
    pri = pri_ref[0, 0]
    vmax = vmax_ref[...]
    gv = jnp.max(vmax)
    wid = lax.broadcasted_iota(jnp.int32, (NW, 1), 0)
    gw = jnp.max(jnp.where(vmax[:, 0:1] == gv, wid, -1))
    hit = wid == gw
    rows = rows_ref[...]
    rowbest = jnp.sum(jnp.where(hit, rows, 0.0), axis=0, keepdims=True)
    use_x = jnp.full((1, F), pri >= gv)
    out_row_ref[...] = jnp.where(use_x, x_ref[...], rowbest)


_tc_select = pl.pallas_call(
    _tc_select_body,
    out_shape=jax.ShapeDtypeStruct((1, F), jnp.float32),
)


def kernel(x, space, memory, index):
    idx_flat = index.reshape(L)
    mem2d = memory.reshape(L, F)
    pri = _tc_priority(x)
    rows, vmaxs = _sc_argmax_rows(idx_flat, mem2d)
    out_row = _tc_select(x, pri, rows, vmaxs)
    return out_row, pri
